# Initial kernel scaffold; baseline (speedup 1.0000x reference)
import jax
import jax.numpy as jnp
from jax.experimental import pallas as pl


def kernel(local_energies, Z, batch, scale, shift):
    raise NotImplementedError("write your pallas kernel here")


# speedup vs baseline: 286.7360x; 286.7360x over previous
import functools

import jax
import jax.numpy as jnp
from jax import lax
from jax.experimental import pallas as pl
from jax.experimental.pallas import tpu as pltpu
from jax.experimental.pallas import tpu_sc as plsc

N = 6_400_000
N_STRUCTURES = 16384
N_SPECIES_PAD = 128
NC, NS = 2, 16
NW = NC * NS
CHUNK = N // NW
BLK = 8000
L = 16


def _sc_body(le_hbm, z_hbm, b_hbm, scale_hbm, shift_hbm, out_hbm,
             scale_v, shift_v, le_v, z_v, b_v, acc_v):
    c = lax.axis_index("c")
    s = lax.axis_index("s")
    wid = s * NC + c
    base = wid * CHUNK

    pltpu.sync_copy(scale_hbm, scale_v)
    pltpu.sync_copy(shift_hbm, shift_v)

    zeros16 = jnp.zeros((L,), jnp.float32)

    def zero_body(i, carry):
        acc_v[pl.ds(i * L, L)] = zeros16
        return carry

    lax.fori_loop(0, N_STRUCTURES // L, zero_body, 0, unroll=4)

    def block_body(g, carry):
        off = base + g * BLK
        pltpu.sync_copy(le_hbm.at[pl.ds(off, BLK)], le_v)
        pltpu.sync_copy(z_hbm.at[pl.ds(off, BLK)], z_v)
        pltpu.sync_copy(b_hbm.at[pl.ds(off, BLK)], b_v)

        def vec_body(j, carry2):
            zz = z_v[pl.ds(j * L, L)]
            sc = plsc.load_gather(scale_v, [zz])
            sh = plsc.load_gather(shift_v, [zz])
            e = le_v[pl.ds(j * L, L)] * sc + sh
            bb = b_v[pl.ds(j * L, L)]
            plsc.addupdate_scatter(acc_v, [bb], e)
            return carry2

        lax.fori_loop(0, BLK // L, vec_body, 0, unroll=1)
        return carry

    lax.fori_loop(0, CHUNK // BLK, block_body, 0)

    pltpu.sync_copy(acc_v, out_hbm.at[wid])


@functools.partial(
    pl.kernel,
    out_type=jax.ShapeDtypeStruct((NW, N_STRUCTURES), jnp.float32),
    mesh=plsc.VectorSubcoreMesh(core_axis_name="c", subcore_axis_name="s"),
    scratch_types=[
        pltpu.VMEM((N_SPECIES_PAD,), jnp.float32),
        pltpu.VMEM((N_SPECIES_PAD,), jnp.float32),
        pltpu.VMEM((BLK,), jnp.float32),
        pltpu.VMEM((BLK,), jnp.int32),
        pltpu.VMEM((BLK,), jnp.int32),
        pltpu.VMEM((N_STRUCTURES,), jnp.float32),
    ],
    compiler_params=pltpu.CompilerParams(needs_layout_passes=False),
)
def _sc_partial_sums(*args):
    _sc_body(*args)


def _merge_body(parts_ref, out_ref):
    out_ref[...] = jnp.sum(parts_ref[...], axis=0)


def kernel(local_energies, Z, batch, scale, shift):
    scale_p = jnp.zeros((N_SPECIES_PAD,), jnp.float32).at[: scale.shape[0]].set(scale)
    shift_p = jnp.zeros((N_SPECIES_PAD,), jnp.float32).at[: shift.shape[0]].set(shift)
    parts = _sc_partial_sums(local_energies, Z, batch, scale_p, shift_p)
    total = pl.pallas_call(
        _merge_body,
        out_shape=jax.ShapeDtypeStruct((N_STRUCTURES,), jnp.float32),
    )(parts)
    return total
